# baseline (speedup 1.0000x reference)
import functools

import jax
import jax.numpy as jnp
from jax import lax
from jax.experimental import pallas as pl
from jax.experimental.pallas import tpu as pltpu
from jax.experimental.pallas import tpu_sc as plsc

B = 16384
K = 32
NROWS = 1000000

_NC = 2
_NS = 16
_NW = _NC * _NS
_BPW = B // _NW
_CH = 128
_NCHUNK = _BPW // _CH


@functools.cache
def _make_gather_sc():
    mesh = plsc.VectorSubcoreMesh(core_axis_name="c", subcore_axis_name="s")

    @functools.partial(
        pl.kernel,
        mesh=mesh,
        compiler_params=pltpu.CompilerParams(
            use_tc_tiling_on_sc=False, needs_layout_passes=False),
        out_type=[
            jax.ShapeDtypeStruct((B, K), jnp.float32),
            jax.ShapeDtypeStruct((B, K), jnp.float32),
        ],
        scratch_types=[
            pltpu.VMEM((_BPW,), jnp.int32),
            pltpu.VMEM((_BPW,), jnp.int32),
            pltpu.VMEM((_BPW * K,), jnp.float32),
            pltpu.VMEM((_BPW * K,), jnp.float32),
            pltpu.VMEM((_BPW, K), jnp.float32),
            pltpu.VMEM((_BPW, K), jnp.float32),
            pltpu.SemaphoreType.DMA,
            pltpu.SemaphoreType.DMA,
        ],
    )
    def _gather_sc(wt_hbm, ht_hbm, ui_hbm, vi_hbm, u_out, v_out,
                   ui_v, vi_v, u_fm, v_fm, u_rows, v_rows, su, sv):
        wid = lax.axis_index("s") * _NC + lax.axis_index("c")
        base = wid * _BPW
        pltpu.sync_copy(ui_hbm.at[wid], ui_v)
        pltpu.sync_copy(vi_hbm.at[wid], vi_v)
        copies = []
        for k in range(K):
            for q in range(_NCHUNK):
                dst = pl.ds(k * _BPW + q * _CH, _CH)
                idx = pl.ds(q * _CH, _CH)
                copies.append(pltpu.async_copy(
                    wt_hbm.at[k].at[ui_v.at[idx]], u_fm.at[dst], su))
                copies.append(pltpu.async_copy(
                    ht_hbm.at[k].at[vi_v.at[idx]], v_fm.at[dst], sv))
        for c in copies:
            c.wait()

        lane = jax.lax.iota(jnp.int32, 16)

        def _tbody(k, _):
            kcol = jnp.full((16,), 0, jnp.int32) + k
            for g in range(_BPW // 16):
                uvec = u_fm[pl.ds(k * _BPW + g * 16, 16)]
                vvec = v_fm[pl.ds(k * _BPW + g * 16, 16)]
                ridx = lane + (g * 16)
                plsc.store_scatter(u_rows, [ridx, kcol], uvec)
                plsc.store_scatter(v_rows, [ridx, kcol], vvec)
            return _

        jax.lax.fori_loop(0, K, _tbody, None)

        pltpu.sync_copy(u_rows, u_out.at[pl.ds(base, _BPW)])
        pltpu.sync_copy(v_rows, v_out.at[pl.ds(base, _BPW)])

    return _gather_sc


def _mlp_body(u_ref, v_ref, a_ref, c_ref, w2_ref, b2_ref, o_ref):
    h = jnp.dot(u_ref[...], a_ref[...], preferred_element_type=jnp.float32)
    h = h + jnp.dot(v_ref[...], c_ref[...], preferred_element_type=jnp.float32)
    h = jnp.maximum(h, 0.0)
    logit = jnp.sum(h * w2_ref[...], axis=1) + b2_ref[...]
    o_ref[...] = jax.nn.sigmoid(logit)


def _mlp_tc(u, v, a, c, w2, b2):
    return pl.pallas_call(
        _mlp_body,
        out_shape=jax.ShapeDtypeStruct((B,), jnp.float32),
    )(u, v, a, c, w2, b2)


def kernel(x, W, H, W1, W2, b2):
    ui = x[:, 0].astype(jnp.int32).reshape(_NW, _BPW)
    vi = x[:, 1].astype(jnp.int32).reshape(_NW, _BPW)
    wt = jnp.transpose(W)
    ht = jnp.transpose(H)
    u, v = _make_gather_sc()(wt, ht, ui, vi)
    a = jnp.transpose(W1[:, :K])
    c = jnp.transpose(W1[:, K:])
    return _mlp_tc(u, v, a, c, W2, b2)


# speedup vs baseline: 21.6246x; 21.6246x over previous
import functools

import jax
import jax.numpy as jnp
from jax import lax
from jax.experimental import pallas as pl
from jax.experimental.pallas import tpu as pltpu
from jax.experimental.pallas import tpu_sc as plsc

B = 16384
K = 32
NROWS = 1000000

_NC = 2
_NS = 16
_NW = _NC * _NS
_BPW = B // _NW
_RING = 4


@functools.cache
def _make_gather_sc():
    mesh = plsc.VectorSubcoreMesh(core_axis_name="c", subcore_axis_name="s")

    @functools.partial(
        pl.kernel,
        mesh=mesh,
        compiler_params=pltpu.CompilerParams(
            use_tc_tiling_on_sc=True, needs_layout_passes=False),
        out_type=[
            jax.ShapeDtypeStruct((K, B), jnp.float32),
            jax.ShapeDtypeStruct((K, B), jnp.float32),
        ],
        scratch_types=(
            [pltpu.VMEM((_BPW,), jnp.int32),
             pltpu.VMEM((_BPW,), jnp.int32),
             pltpu.VMEM((K, _BPW), jnp.float32),
             pltpu.VMEM((K, _BPW), jnp.float32)]
            + [pltpu.VMEM((K, 128), jnp.float32)] * (2 * _RING)
            + [pltpu.SemaphoreType.DMA] * (2 * _RING)
        ),
    )
    def _gather_sc(wt_hbm, ht_hbm, ui_hbm, vi_hbm, ut_out, vt_out,
                   ui_v, vi_v, u_fm, v_fm, *ring):
        ublk = ring[0:_RING]
        vblk = ring[_RING:2 * _RING]
        usem = ring[2 * _RING:3 * _RING]
        vsem = ring[3 * _RING:4 * _RING]

        wid = lax.axis_index("s") * _NC + lax.axis_index("c")
        base = wid * _BPW
        pltpu.sync_copy(ui_hbm.at[pl.ds(base, _BPW)], ui_v)
        pltpu.sync_copy(vi_hbm.at[pl.ds(base, _BPW)], vi_v)

        lane = jax.lax.iota(jnp.int32, 16)
        rows_lo = lane
        rows_hi = lane + 16

        def _start(j, r, s):
            slot = j % _RING
            c0u = pl.multiple_of(r - (r % 128), 128)
            c0v = pl.multiple_of(s - (s % 128), 128)
            pltpu.make_async_copy(
                wt_hbm.at[:, pl.ds(c0u, 128)], ublk[slot], usem[slot]
            ).start()
            pltpu.make_async_copy(
                ht_hbm.at[:, pl.ds(c0v, 128)], vblk[slot], vsem[slot]
            ).start()

        def _finish(j, r, s, dstcol):
            slot = j % _RING
            pltpu.make_async_copy(
                wt_hbm.at[:, pl.ds(0, 128)], ublk[slot], usem[slot]
            ).wait()
            pltpu.make_async_copy(
                ht_hbm.at[:, pl.ds(0, 128)], vblk[slot], vsem[slot]
            ).wait()
            cu = jnp.full((16,), 0, jnp.int32) + (r % 128)
            cv = jnp.full((16,), 0, jnp.int32) + (s % 128)
            dc = jnp.full((16,), 0, jnp.int32) + dstcol
            for rows in (rows_lo, rows_hi):
                uvecf = plsc.load_gather(ublk[slot], [rows, cu])
                vvecf = plsc.load_gather(vblk[slot], [rows, cv])
                plsc.store_scatter(u_fm, [rows, dc], uvecf)
                plsc.store_scatter(v_fm, [rows, dc], vvecf)

        def _body(g, carry):
            upv, vpv = carry
            uvec = ui_v[pl.ds(g * 16, 16)]
            vvec = vi_v[pl.ds(g * 16, 16)]
            for j in range(16):
                if j < _RING:
                    @pl.when(g >= 1)
                    def _fin():
                        _finish(j, upv[j + 16 - _RING], vpv[j + 16 - _RING],
                                g * 16 + j - _RING)
                else:
                    _finish(j, uvec[j - _RING], vvec[j - _RING],
                            g * 16 + j - _RING)
                _start(j, uvec[j], vvec[j])
            return (uvec, vvec)

        zero16 = jnp.zeros((16,), jnp.int32)
        upv, vpv = lax.fori_loop(0, _BPW // 16, _body, (zero16, zero16))

        for j in range(_RING):
            _finish(j, upv[j + 16 - _RING], vpv[j + 16 - _RING],
                    _BPW + j - _RING)

        pltpu.sync_copy(u_fm, ut_out.at[:, pl.ds(base, _BPW)])
        pltpu.sync_copy(v_fm, vt_out.at[:, pl.ds(base, _BPW)])

    return _gather_sc


def _mlp_body(ut_ref, vt_ref, w1a_ref, w1b_ref, w2_ref, b2_ref, o_ref):
    ht = jnp.dot(w1a_ref[...], ut_ref[...], preferred_element_type=jnp.float32)
    ht = ht + jnp.dot(w1b_ref[...], vt_ref[...],
                      preferred_element_type=jnp.float32)
    ht = jnp.maximum(ht, 0.0)
    logit = jnp.dot(w2_ref[...], ht, preferred_element_type=jnp.float32)
    o_ref[...] = jax.nn.sigmoid(logit + b2_ref[...])


def _mlp_tc(ut, vt, w1a, w1b, w2, b2):
    return pl.pallas_call(
        _mlp_body,
        out_shape=jax.ShapeDtypeStruct((1, B), jnp.float32),
    )(ut, vt, w1a, w1b, w2, b2)


def kernel(x, W, H, W1, W2, b2):
    ui = x[:, 0].astype(jnp.int32)
    vi = x[:, 1].astype(jnp.int32)
    wt = jnp.transpose(W)
    ht = jnp.transpose(H)
    ut, vt = _make_gather_sc()(wt, ht, ui, vi)
    w1a = W1[:, :K]
    w1b = W1[:, K:]
    out = _mlp_tc(ut, vt, w1a, w1b, W2, b2.reshape(1, 1))
    return out.reshape(B)


# speedup vs baseline: 22.0446x; 1.0194x over previous
import functools

import jax
import jax.numpy as jnp
from jax import lax
from jax.experimental import pallas as pl
from jax.experimental.pallas import tpu as pltpu
from jax.experimental.pallas import tpu_sc as plsc

B = 16384
K = 32
NROWS = 1000000

_NC = 2
_NS = 16
_NW = _NC * _NS
_BPW = B // _NW
_RING = 8


@functools.cache
def _make_gather_sc():
    mesh = plsc.VectorSubcoreMesh(core_axis_name="c", subcore_axis_name="s")

    @functools.partial(
        pl.kernel,
        mesh=mesh,
        compiler_params=pltpu.CompilerParams(
            use_tc_tiling_on_sc=True, needs_layout_passes=False),
        out_type=[
            jax.ShapeDtypeStruct((K, B), jnp.float32),
            jax.ShapeDtypeStruct((K, B), jnp.float32),
        ],
        scratch_types=(
            [pltpu.VMEM((_BPW,), jnp.int32),
             pltpu.VMEM((_BPW,), jnp.int32),
             pltpu.VMEM((K, _BPW), jnp.float32),
             pltpu.VMEM((K, _BPW), jnp.float32)]
            + [pltpu.VMEM((K, 128), jnp.float32)] * (2 * _RING)
            + [pltpu.SemaphoreType.DMA] * (2 * _RING)
        ),
    )
    def _gather_sc(wt_hbm, ht_hbm, ui_hbm, vi_hbm, ut_out, vt_out,
                   ui_v, vi_v, u_fm, v_fm, *ring):
        ublk = ring[0:_RING]
        vblk = ring[_RING:2 * _RING]
        usem = ring[2 * _RING:3 * _RING]
        vsem = ring[3 * _RING:4 * _RING]

        wid = lax.axis_index("s") * _NC + lax.axis_index("c")
        base = wid * _BPW
        pltpu.sync_copy(ui_hbm.at[pl.ds(base, _BPW)], ui_v)
        pltpu.sync_copy(vi_hbm.at[pl.ds(base, _BPW)], vi_v)

        lane = jax.lax.iota(jnp.int32, 16)
        rows_lo = lane
        rows_hi = lane + 16

        def _start(j, r, s):
            slot = j % _RING
            c0u = pl.multiple_of(r - (r % 128), 128)
            c0v = pl.multiple_of(s - (s % 128), 128)
            pltpu.make_async_copy(
                wt_hbm.at[:, pl.ds(c0u, 128)], ublk[slot], usem[slot]
            ).start()
            pltpu.make_async_copy(
                ht_hbm.at[:, pl.ds(c0v, 128)], vblk[slot], vsem[slot]
            ).start()

        def _finish(j, r, s, dstcol):
            slot = j % _RING
            pltpu.make_async_copy(
                wt_hbm.at[:, pl.ds(0, 128)], ublk[slot], usem[slot]
            ).wait()
            pltpu.make_async_copy(
                ht_hbm.at[:, pl.ds(0, 128)], vblk[slot], vsem[slot]
            ).wait()
            cu = jnp.full((16,), 0, jnp.int32) + (r % 128)
            cv = jnp.full((16,), 0, jnp.int32) + (s % 128)
            dc = jnp.full((16,), 0, jnp.int32) + dstcol
            for rows in (rows_lo, rows_hi):
                uvecf = plsc.load_gather(ublk[slot], [rows, cu])
                vvecf = plsc.load_gather(vblk[slot], [rows, cv])
                plsc.store_scatter(u_fm, [rows, dc], uvecf)
                plsc.store_scatter(v_fm, [rows, dc], vvecf)

        def _body(g, carry):
            upv, vpv = carry
            uvec = ui_v[pl.ds(g * 16, 16)]
            vvec = vi_v[pl.ds(g * 16, 16)]
            for j in range(16):
                if j < _RING:
                    @pl.when(g >= 1)
                    def _fin():
                        _finish(j, upv[j + 16 - _RING], vpv[j + 16 - _RING],
                                g * 16 + j - _RING)
                else:
                    _finish(j, uvec[j - _RING], vvec[j - _RING],
                            g * 16 + j - _RING)
                _start(j, uvec[j], vvec[j])
            return (uvec, vvec)

        zero16 = jnp.zeros((16,), jnp.int32)
        upv, vpv = lax.fori_loop(0, _BPW // 16, _body, (zero16, zero16))

        for j in range(_RING):
            _finish(j, upv[j + 16 - _RING], vpv[j + 16 - _RING],
                    _BPW + j - _RING)

        pltpu.sync_copy(u_fm, ut_out.at[:, pl.ds(base, _BPW)])
        pltpu.sync_copy(v_fm, vt_out.at[:, pl.ds(base, _BPW)])

    return _gather_sc


def _mlp_body(ut_ref, vt_ref, w1a_ref, w1b_ref, w2_ref, b2_ref, o_ref):
    ht = jnp.dot(w1a_ref[...], ut_ref[...], preferred_element_type=jnp.float32)
    ht = ht + jnp.dot(w1b_ref[...], vt_ref[...],
                      preferred_element_type=jnp.float32)
    ht = jnp.maximum(ht, 0.0)
    logit = jnp.dot(w2_ref[...], ht, preferred_element_type=jnp.float32)
    o_ref[...] = jax.nn.sigmoid(logit + b2_ref[...])


def _mlp_tc(ut, vt, w1a, w1b, w2, b2):
    return pl.pallas_call(
        _mlp_body,
        out_shape=jax.ShapeDtypeStruct((1, B), jnp.float32),
    )(ut, vt, w1a, w1b, w2, b2)


def kernel(x, W, H, W1, W2, b2):
    ui = x[:, 0].astype(jnp.int32)
    vi = x[:, 1].astype(jnp.int32)
    wt = jnp.transpose(W)
    ht = jnp.transpose(H)
    ut, vt = _make_gather_sc()(wt, ht, ui, vi)
    w1a = W1[:, :K]
    w1b = W1[:, K:]
    out = _mlp_tc(ut, vt, w1a, w1b, W2, b2.reshape(1, 1))
    return out.reshape(B)
